# baseline (speedup 1.0000x reference)
import functools

import jax
import jax.numpy as jnp
from jax import lax
from jax.experimental import pallas as pl
from jax.experimental.pallas import tpu as pltpu
from jax.experimental.pallas import tpu_sc as plsc

_L = 200
_DIM = 64
_PAD = 256
_NSUB = 16
_NCHUNK = _DIM // 16


_SCRATCH = [
    pltpu.VMEM((2 * _PAD,), jnp.int32),
    pltpu.VMEM((16,), jnp.int32),
    pltpu.VMEM((16, 2 * _DIM), jnp.float32),
    pltpu.VMEM((_NSUB * _DIM,), jnp.float32),
    pltpu.VMEM((_DIM,), jnp.float32),
    pltpu.VMEM_SHARED((_NSUB * _DIM,), jnp.float32),
    pltpu.SemaphoreType.DMA,
]


def _emb_agg_body(idx_hbm, table2_hbm, out_hbm,
                  idx_all_v, row_v, rows_v, red_v, vec_v, shared, sem):
    cid = lax.axis_index("c")
    sid = lax.axis_index("s")
    base = cid * _PAD + sid * 16

    pltpu.sync_copy(idx_hbm, idx_all_v)
    row_v[...] = idx_all_v[pl.ds(base, 16)]

    pltpu.async_copy(table2_hbm.at[row_v], rows_v, sem).wait()

    acc = [jnp.zeros((16,), jnp.float32) for _ in range(_NCHUNK)]
    for j in range(16):
        valid = jnp.where(sid * 16 + j < _L, jnp.float32(1.0), jnp.float32(0.0))
        for c in range(_NCHUNK):
            acc[c] = acc[c] + rows_v[j, pl.ds(c * 16, 16)] * valid
    for c in range(_NCHUNK):
        vec_v[pl.ds(c * 16, 16)] = acc[c]

    pltpu.sync_copy(vec_v, shared.at[pl.ds(sid * _DIM, _DIM)])
    plsc.subcore_barrier()

    @pl.when(sid == 0)
    def _reduce():
        pltpu.sync_copy(shared, red_v)
        tot = [jnp.zeros((16,), jnp.float32) for _ in range(_NCHUNK)]
        for r in range(_NSUB):
            for c in range(_NCHUNK):
                tot[c] = tot[c] + red_v[pl.ds(r * _DIM + c * 16, 16)]
        inv = jnp.float32(1.0 / _L)
        for c in range(_NCHUNK):
            vec_v[pl.ds(c * 16, 16)] = tot[c] * inv
        pltpu.sync_copy(vec_v, out_hbm.at[cid])


_emb_agg = pl.kernel(
    _emb_agg_body,
    out_type=jax.ShapeDtypeStruct((2, _DIM), jnp.float32),
    scratch_types=_SCRATCH,
    mesh=plsc.VectorSubcoreMesh(core_axis_name="c", subcore_axis_name="s"),
    compiler_params=pltpu.CompilerParams(needs_layout_passes=False),
)


def _repack_body(x_ref, o_ref):
    o_ref[:, : _DIM] = x_ref[...]
    o_ref[:, _DIM:] = jnp.zeros_like(x_ref[...])


def _repack(table):
    v = table.shape[0]
    blk = 2000
    return pl.pallas_call(
        _repack_body,
        grid=(v // blk,),
        in_specs=[pl.BlockSpec((blk, _DIM), lambda i: (i, 0))],
        out_specs=pl.BlockSpec((blk, 2 * _DIM), lambda i: (i, 0)),
        out_shape=jax.ShapeDtypeStruct((v, 2 * _DIM), jnp.float32),
    )(table)


def kernel(s1, s2, table):
    table2 = _repack(table)
    pad = jnp.zeros((_PAD - _L,), jnp.int32)
    idx = jnp.concatenate([s1.astype(jnp.int32), pad,
                           s2.astype(jnp.int32), pad])
    return _emb_agg(idx, table2).reshape(2 * _DIM)


# speedup vs baseline: 1.8239x; 1.8239x over previous
import functools

import jax
import jax.numpy as jnp
from jax import lax
from jax.experimental import pallas as pl
from jax.experimental.pallas import tpu as pltpu
from jax.experimental.pallas import tpu_sc as plsc

_L = 200
_PAD = 256
_DIM = 64
_BLK = 2048
_VOCAB = 100000
_NSTEP = -(-_VOCAB // _BLK)
_VP = _NSTEP * _BLK


def _wbuild_body(idx_hbm, wts_hbm, zeros_hbm, w2_hbm, acc_v, idx_v, wts_v):
    cid = lax.axis_index("c")
    sid = lax.axis_index("s")

    @pl.when(sid == 0)
    def _():
        pltpu.sync_copy(zeros_hbm, acc_v)
        pltpu.sync_copy(idx_hbm, idx_v)
        pltpu.sync_copy(wts_hbm, wts_v)
        base = cid * _PAD
        for k in range(_PAD // 16):
            iv = idx_v[pl.ds(base + k * 16, 16)]
            wv = wts_v[pl.ds(base + k * 16, 16)]
            plsc.addupdate_scatter(acc_v, [iv], wv)
        pltpu.sync_copy(acc_v, w2_hbm.at[cid])


_wbuild = pl.kernel(
    _wbuild_body,
    out_type=jax.ShapeDtypeStruct((2, _VP), jnp.float32),
    scratch_types=[
        pltpu.VMEM((_VP,), jnp.float32),
        pltpu.VMEM((2 * _PAD,), jnp.int32),
        pltpu.VMEM((2 * _PAD,), jnp.float32),
    ],
    mesh=plsc.VectorSubcoreMesh(core_axis_name="c", subcore_axis_name="s"),
    compiler_params=pltpu.CompilerParams(needs_layout_passes=False),
)


def _wsum_body(x_ref, w_ref, o_ref, acc1, acc2):
    pid = pl.program_id(0)

    @pl.when(pid == 0)
    def _():
        acc1[...] = jnp.zeros_like(acc1)
        acc2[...] = jnp.zeros_like(acc2)

    col = jax.lax.broadcasted_iota(jnp.int32, (1, _BLK), 1) + pid * _BLK
    x = jnp.where(col < _VOCAB, x_ref[...], 0.0)
    acc1[...] += x * w_ref[0:1, :]
    acc2[...] += x * w_ref[1:2, :]

    @pl.when(pid == _NSTEP - 1)
    def _():
        o_ref[0, :] = jnp.sum(acc1[...], axis=1)
        o_ref[1, :] = jnp.sum(acc2[...], axis=1)


def _wsum(table_t, w2):
    return pl.pallas_call(
        _wsum_body,
        grid=(_NSTEP,),
        in_specs=[
            pl.BlockSpec((_DIM, _BLK), lambda i: (0, i)),
            pl.BlockSpec((2, _BLK), lambda i: (0, i)),
        ],
        out_specs=pl.BlockSpec((2, _DIM), lambda i: (0, 0)),
        out_shape=jax.ShapeDtypeStruct((2, _DIM), jnp.float32),
        scratch_shapes=[
            pltpu.VMEM((_DIM, _BLK), jnp.float32),
            pltpu.VMEM((_DIM, _BLK), jnp.float32),
        ],
    )(table_t, w2)


def kernel(s1, s2, table):
    pad = jnp.zeros((_PAD - _L,), jnp.int32)
    idx = jnp.concatenate([s1.astype(jnp.int32), pad,
                           s2.astype(jnp.int32), pad])
    pos = jnp.arange(_PAD, dtype=jnp.int32)
    wts1 = jnp.where(pos < _L, jnp.float32(1.0 / _L), jnp.float32(0.0))
    wts = jnp.concatenate([wts1, wts1])
    zeros = jnp.zeros((_VP,), jnp.float32)
    w2 = _wbuild(idx, wts, zeros)
    table_t = jnp.transpose(table)
    out = _wsum(table_t, w2)
    return out.reshape(2 * _DIM)
